# baseline (speedup 1.0000x reference)
import functools
import math

import jax
import jax.numpy as jnp
import numpy as np
from jax.experimental import pallas as pl

ROWS_PER_FRAME = 543
N_FRAMES = 4096
MAX_LEN = 2048
K = ROWS_PER_FRAME * 3

_REF = [500, 501, 512, 513, 159, 386, 13]
_LIP = [61, 185, 40, 39, 37, 0, 267, 269, 270, 409, 291, 146, 91, 181, 84,
        17, 314, 405, 321, 375, 78, 191, 80, 81, 82, 13, 312, 311, 310, 415,
        95, 88, 178, 87, 14, 317, 402, 318, 324, 308]
_LLIP = _LIP[10::-1] + _LIP[19:10:-1] + _LIP[29:19:-1] + _LIP[39:29:-1]
_LHAND = list(range(468, 489))
_RHAND = list(range(522, 543))

_SEL_R = _LIP + _RHAND
_SEL_L = _LLIP + _LHAND

OUTC = 122


def _build_constants():
    P = np.zeros((K, 256), np.float32)
    Pabs = np.zeros((K, 256), np.float32)
    sign = np.zeros((1, 256), np.float32)
    csel = np.zeros((1, 256), np.float32)
    for j, lm in enumerate(_SEL_R):
        for c in range(2):
            col = 2 * j + c
            P[3 * lm + c, col] = 1.0
            for cc in range(3):
                Pabs[3 * lm + cc, col] = 1.0
            sign[0, col] = 1.0
            csel[0, col] = float(c)
    for j, lm in enumerate(_SEL_L):
        for c in range(2):
            col = 128 + 2 * j + c
            v = -1.0 if c == 0 else 1.0
            P[3 * lm + c, col] = v
            for cc in range(3):
                Pabs[3 * lm + cc, col] = 1.0
            sign[0, col] = v
            csel[0, col] = float(c)

    Pstat = np.zeros((K, 32), np.float32)
    for k, lm in enumerate(_REF):
        for c in range(3):
            Pstat[3 * lm + c, 3 * k + c] = 1.0
    for lm in _LHAND:
        for c in range(3):
            Pstat[3 * lm + c, 24] = 1.0
    for lm in _RHAND:
        for c in range(3):
            Pstat[3 * lm + c, 25] = 1.0
    F = np.zeros((32, 8), np.float32)
    F2 = np.zeros((32, 8), np.float32)
    for k in range(7):
        for c in range(3):
            F[3 * k + c, k] = 1.0
            F2[3 * k + c, c] = 1.0
    return (jnp.asarray(P), jnp.asarray(Pabs), jnp.asarray(sign),
            jnp.asarray(csel), jnp.asarray(Pstat), jnp.asarray(F),
            jnp.asarray(F2))


def _stats_body(x_ref, pstat_ref, f_ref, f2_ref, out_ref):
    i = pl.program_id(0)
    x = x_ref[...]
    nanm = jnp.isnan(x)
    nf = nanm.astype(jnp.float32)
    x0 = jnp.where(nanm, 0.0, x)

    gx = jnp.dot(x0, pstat_ref[...], preferred_element_type=jnp.float32)
    gn = jnp.dot(nf, pstat_ref[...], preferred_element_type=jnp.float32)

    rowcnt = jnp.dot(gn, f_ref[...], preferred_element_type=jnp.float32)
    w = (rowcnt == 0.0).astype(jnp.float32)
    wexp = jnp.dot(w, f_ref[...].T, preferred_element_type=jnp.float32)

    gxm = gx * wexp
    gx2m = gx * gx * wexp
    sums3 = jnp.dot(jnp.sum(gxm, axis=0, keepdims=True), f2_ref[...],
                    preferred_element_type=jnp.float32)
    sumsq3 = jnp.dot(jnp.sum(gx2m, axis=0, keepdims=True), f2_ref[...],
                     preferred_element_type=jnp.float32)
    cnt = jnp.sum(w[:, :7])

    lflag = (gn[:, 24] == 0.0).astype(jnp.float32)
    rflag = (gn[:, 25] == 0.0).astype(jnp.float32)
    lcnt = jnp.sum(lflag)
    rcnt = jnp.sum(rflag)

    lane = jax.lax.broadcasted_iota(jnp.int32, (1, 128), 1)
    part = jnp.zeros((1, 128), jnp.float32)
    part = jnp.where(lane == 0, cnt, part)
    for c in range(3):
        part = jnp.where(lane == 1 + c, sums3[0, c], part)
        part = jnp.where(lane == 4 + c, sumsq3[0, c], part)
    part = jnp.where(lane == 7, lcnt, part)
    part = jnp.where(lane == 8, rcnt, part)

    @pl.when(i == 0)
    def _():
        out_ref[...] = jnp.zeros_like(out_ref)

    out_ref[...] += part


def _main_body(part_ref, x_ref, p_ref, pabs_ref, sign_ref, csel_ref, out_ref):
    p = part_ref[...]
    cnt = p[0, 0]
    m0 = p[0, 1] / cnt
    m1 = p[0, 2] / cnt
    m2 = p[0, 3] / cnt
    v0 = p[0, 4] / cnt - m0 * m0
    v1 = p[0, 5] / cnt - m1 * m1
    v2 = p[0, 6] / cnt - m2 * m2
    s = (jnp.sqrt(v0) + jnp.sqrt(v1) + jnp.sqrt(v2)) / 3.0
    lhanded = p[0, 7] > p[0, 8]

    x = x_ref[:, 0, :]
    nanm = jnp.isnan(x)
    nf = nanm.astype(jnp.float32)
    x0 = jnp.where(nanm, 0.0, x)

    v = jnp.dot(x0, p_ref[...], preferred_element_type=jnp.float32)
    mk = jnp.dot(nf, pabs_ref[...], preferred_element_type=jnp.float32)

    val = jnp.where(lhanded, v[:, 128:256], v[:, 0:128])
    bad = jnp.where(lhanded, mk[:, 128:256], mk[:, 0:128]) > 0.5
    sgn = jnp.where(lhanded, sign_ref[0, 128:256], sign_ref[0, 0:128])
    cs = jnp.where(lhanded, csel_ref[0, 128:256], csel_ref[0, 0:128])

    m_vec = m0 + (m1 - m0) * cs
    res = (val - sgn * m_vec[None, :]) / s
    res = jnp.where(bad, 0.0, res)
    out_ref[...] = res[:, :OUTC]


@jax.jit
def kernel(frames):
    (P, Pabs, sign, csel, Pstat, F, F2) = _build_constants()
    x2d = frames.reshape(N_FRAMES, K)

    GA = 512
    partials = pl.pallas_call(
        _stats_body,
        grid=(N_FRAMES // GA,),
        in_specs=[
            pl.BlockSpec((GA, K), lambda i: (i, 0)),
            pl.BlockSpec((K, 32), lambda i: (0, 0)),
            pl.BlockSpec((32, 8), lambda i: (0, 0)),
            pl.BlockSpec((32, 8), lambda i: (0, 0)),
        ],
        out_specs=pl.BlockSpec((1, 128), lambda i: (0, 0)),
        out_shape=jax.ShapeDtypeStruct((1, 128), jnp.float32),
    )(x2d, Pstat, F, F2)

    GB = 256
    x3d = frames.reshape(MAX_LEN, 2, K)
    out2d = pl.pallas_call(
        _main_body,
        grid=(MAX_LEN // GB,),
        in_specs=[
            pl.BlockSpec((1, 128), lambda i: (0, 0)),
            pl.BlockSpec((GB, 2, K), lambda i: (i, 0, 0)),
            pl.BlockSpec((K, 256), lambda i: (0, 0)),
            pl.BlockSpec((K, 256), lambda i: (0, 0)),
            pl.BlockSpec((1, 256), lambda i: (0, 0)),
            pl.BlockSpec((1, 256), lambda i: (0, 0)),
        ],
        out_specs=pl.BlockSpec((GB, OUTC), lambda i: (i, 0)),
        out_shape=jax.ShapeDtypeStruct((MAX_LEN, OUTC), jnp.float32),
    )(partials, x3d, P, Pabs, sign, csel)

    return out2d.reshape(MAX_LEN, 61, 2)


# speedup vs baseline: 114.5981x; 114.5981x over previous
import jax
import jax.numpy as jnp
import numpy as np
from jax.experimental import pallas as pl

ROWS_PER_FRAME = 543
N_FRAMES = 4096
MAX_LEN = 2048

_REF = [500, 501, 512, 513, 159, 386, 13]
_LIP = [61, 185, 40, 39, 37, 0, 267, 269, 270, 409, 291, 146, 91, 181, 84,
        17, 314, 405, 321, 375, 78, 191, 80, 81, 82, 13, 312, 311, 310, 415,
        95, 88, 178, 87, 14, 317, 402, 318, 324, 308]
_LLIP = _LIP[10::-1] + _LIP[19:10:-1] + _LIP[29:19:-1] + _LIP[39:29:-1]
_LHAND = list(range(468, 489))
_RHAND = list(range(522, 543))

_SEL_R = _LIP + _RHAND
_SEL_L = _LLIP + _LHAND


def _build_sel():
    S0 = np.zeros((128, ROWS_PER_FRAME), np.float32)
    S1 = np.zeros((128, ROWS_PER_FRAME), np.float32)
    Sm = np.zeros((128, ROWS_PER_FRAME), np.float32)
    for j, lm in enumerate(_SEL_R):
        S0[j, lm] = 1.0
        S1[j, lm] = 1.0
        Sm[j, lm] = 1.0
    for j, lm in enumerate(_SEL_L):
        S0[64 + j, lm] = -1.0
        S1[64 + j, lm] = 1.0
        Sm[64 + j, lm] = 1.0
    return jnp.asarray(S0), jnp.asarray(S1), jnp.asarray(Sm)


def _stats_body(x_ref, out_ref):
    i = pl.program_id(0)
    x = x_ref[...]
    isn = jnp.isnan(x)
    nanany = isn[0] | isn[1] | isn[2]
    x0 = jnp.where(isn, 0.0, x)

    cnt = 0.0
    sums = [0.0, 0.0, 0.0]
    sumsq = [0.0, 0.0, 0.0]
    for lm in _REF:
        w = 1.0 - nanany[lm, :].astype(jnp.float32)
        cnt = cnt + jnp.sum(w)
        for c in range(3):
            v = x0[c, lm, :] * w
            sums[c] = sums[c] + jnp.sum(v)
            sumsq[c] = sumsq[c] + jnp.sum(v * x0[c, lm, :])

    lbad = jnp.any(nanany[468:489, :], axis=0)
    rbad = jnp.any(nanany[522:543, :], axis=0)
    lcnt = jnp.sum(1.0 - lbad.astype(jnp.float32))
    rcnt = jnp.sum(1.0 - rbad.astype(jnp.float32))

    lane = jax.lax.broadcasted_iota(jnp.int32, (1, 128), 1)
    part = jnp.zeros((1, 128), jnp.float32)
    part = jnp.where(lane == 0, cnt, part)
    for c in range(3):
        part = jnp.where(lane == 1 + c, sums[c], part)
        part = jnp.where(lane == 4 + c, sumsq[c], part)
    part = jnp.where(lane == 7, lcnt, part)
    part = jnp.where(lane == 8, rcnt, part)

    @pl.when(i == 0)
    def _():
        out_ref[...] = jnp.zeros_like(out_ref)

    out_ref[...] += part


def _main_body(part_ref, x_ref, s0_ref, s1_ref, sm_ref, out_ref):
    p = part_ref[...]
    cnt = p[0, 0]
    m0 = p[0, 1] / cnt
    m1 = p[0, 2] / cnt
    m2 = p[0, 3] / cnt
    v0 = p[0, 4] / cnt - m0 * m0
    v1 = p[0, 5] / cnt - m1 * m1
    v2 = p[0, 6] / cnt - m2 * m2
    inv_s = 3.0 / (jnp.sqrt(v0) + jnp.sqrt(v1) + jnp.sqrt(v2))
    lhanded = p[0, 7] > p[0, 8]

    x = x_ref[...]
    isn = jnp.isnan(x)
    nanany = (isn[0] | isn[1] | isn[2]).astype(jnp.float32)
    x0 = jnp.where(isn, 0.0, x)

    va = jnp.dot(s0_ref[...], x0[0], preferred_element_type=jnp.float32)
    vb = jnp.dot(s1_ref[...], x0[1], preferred_element_type=jnp.float32)
    mk = jnp.dot(sm_ref[...], nanany, preferred_element_type=jnp.float32)

    val0 = jnp.where(lhanded, va[64:128, :], va[0:64, :])
    val1 = jnp.where(lhanded, vb[64:128, :], vb[0:64, :])
    bad = jnp.where(lhanded, mk[64:128, :], mk[0:64, :]) > 0.5
    sgn0 = jnp.where(lhanded, -1.0, 1.0)

    r0 = (val0 - sgn0 * m0) * inv_s
    r1 = (val1 - m1) * inv_s
    r0 = jnp.where(bad, 0.0, r0)
    r1 = jnp.where(bad, 0.0, r1)
    out_ref[0, :, :] = r0
    out_ref[1, :, :] = r1


@jax.jit
def kernel(frames):
    S0, S1, Sm = _build_sel()
    xT = jnp.transpose(frames, (2, 1, 0))

    FA = 512
    partials = pl.pallas_call(
        _stats_body,
        grid=(N_FRAMES // FA,),
        in_specs=[pl.BlockSpec((3, ROWS_PER_FRAME, FA), lambda i: (0, 0, i))],
        out_specs=pl.BlockSpec((1, 128), lambda i: (0, 0)),
        out_shape=jax.ShapeDtypeStruct((1, 128), jnp.float32),
    )(xT)

    FB = 512
    full = pl.pallas_call(
        _main_body,
        grid=(N_FRAMES // FB,),
        in_specs=[
            pl.BlockSpec((1, 128), lambda j: (0, 0)),
            pl.BlockSpec((3, ROWS_PER_FRAME, FB), lambda j: (0, 0, j)),
            pl.BlockSpec((128, ROWS_PER_FRAME), lambda j: (0, 0)),
            pl.BlockSpec((128, ROWS_PER_FRAME), lambda j: (0, 0)),
            pl.BlockSpec((128, ROWS_PER_FRAME), lambda j: (0, 0)),
        ],
        out_specs=pl.BlockSpec((2, 64, FB), lambda j: (0, 0, j)),
        out_shape=jax.ShapeDtypeStruct((2, 64, N_FRAMES), jnp.float32),
    )(partials, xT, S0, S1, Sm)

    res = full[:, :61, ::2]
    return jnp.transpose(res, (2, 1, 0))


# speedup vs baseline: 153.3101x; 1.3378x over previous
import jax
import jax.numpy as jnp
import numpy as np
from jax.experimental import pallas as pl

ROWS_PER_FRAME = 543
N_FRAMES = 4096
MAX_LEN = 2048

_REF = [500, 501, 512, 513, 159, 386, 13]
_LIP = [61, 185, 40, 39, 37, 0, 267, 269, 270, 409, 291, 146, 91, 181, 84,
        17, 314, 405, 321, 375, 78, 191, 80, 81, 82, 13, 312, 311, 310, 415,
        95, 88, 178, 87, 14, 317, 402, 318, 324, 308]
_LLIP = _LIP[10::-1] + _LIP[19:10:-1] + _LIP[29:19:-1] + _LIP[39:29:-1]
_LHAND = list(range(468, 489))
_RHAND = list(range(522, 543))

_SEL_R = _LIP + _RHAND
_SEL_L = _LLIP + _LHAND

_RB = 8
_STAT_ROWBLKS = sorted({lm // _RB for lm in _REF + _LHAND + _RHAND})
_NRB = len(_STAT_ROWBLKS)


def _build_sel():
    S0 = np.zeros((128, ROWS_PER_FRAME), np.float32)
    S1 = np.zeros((128, ROWS_PER_FRAME), np.float32)
    Sm = np.zeros((128, ROWS_PER_FRAME), np.float32)
    for j, lm in enumerate(_SEL_R):
        S0[j, lm] = 1.0
        S1[j, lm] = 1.0
        Sm[j, lm] = 1.0
    for j, lm in enumerate(_SEL_L):
        S0[64 + j, lm] = -1.0
        S1[64 + j, lm] = 1.0
        Sm[64 + j, lm] = 1.0
    return jnp.asarray(S0), jnp.asarray(S1), jnp.asarray(Sm)


def _build_stat_masks():
    m = np.zeros((_NRB, 3, _RB), np.float32)
    for i, blk in enumerate(_STAT_ROWBLKS):
        for r in range(_RB):
            lm = blk * _RB + r
            if lm in _REF:
                m[i, 0, r] = 1.0
            if lm in _LHAND:
                m[i, 1, r] = 1.0
            if lm in _RHAND:
                m[i, 2, r] = 1.0
    return jnp.asarray(m)


def _rowblk(rb):
    v = 55 + rb + jnp.where(rb >= 8, 1, 0)
    v = jnp.where(rb == 0, 1, v)
    v = jnp.where(rb == 1, 19, v)
    v = jnp.where(rb == 2, 48, v)
    return v


def _stats_body(x_ref, mask_ref, out_ref, cnt_ref):
    fb = pl.program_id(0)
    rb = pl.program_id(1)
    x = x_ref[...]
    isn = jnp.isnan(x)
    nanany = (isn[0] | isn[1] | isn[2]).astype(jnp.float32)
    x0 = jnp.where(isn, 0.0, x)

    refw = mask_ref[0, 0, :][:, None]
    lw = mask_ref[0, 1, :][:, None]
    rw = mask_ref[0, 2, :][:, None]

    w = refw * (1.0 - nanany)
    cnt = jnp.sum(w)
    sums = []
    sumsq = []
    for c in range(3):
        v = x0[c] * w
        sums.append(jnp.sum(v))
        sumsq.append(jnp.sum(v * x0[c]))

    lane = jax.lax.broadcasted_iota(jnp.int32, (1, 128), 1)
    part = jnp.zeros((1, 128), jnp.float32)
    part = jnp.where(lane == 0, cnt, part)
    for c in range(3):
        part = jnp.where(lane == 1 + c, sums[c], part)
        part = jnp.where(lane == 4 + c, sumsq[c], part)

    @pl.when((fb == 0) & (rb == 0))
    def _():
        out_ref[...] = jnp.zeros_like(out_ref)

    out_ref[...] += part

    lpart = jnp.sum(lw * nanany, axis=0, keepdims=True)
    rpart = jnp.sum(rw * nanany, axis=0, keepdims=True)

    @pl.when(rb == 0)
    def _():
        cnt_ref[...] = jnp.zeros_like(cnt_ref)

    cnt_ref[0:1, :] += lpart
    cnt_ref[1:2, :] += rpart


def _main_body(part_ref, hand_ref, x_ref, s0_ref, s1_ref, sm_ref, e_ref,
               out_ref):
    p = part_ref[...]
    cnt = p[0, 0]
    m0 = p[0, 1] / cnt
    m1 = p[0, 2] / cnt
    m2 = p[0, 3] / cnt
    v0 = p[0, 4] / cnt - m0 * m0
    v1 = p[0, 5] / cnt - m1 * m1
    v2 = p[0, 6] / cnt - m2 * m2
    inv_s = 3.0 / (jnp.sqrt(v0) + jnp.sqrt(v1) + jnp.sqrt(v2))
    hn = hand_ref[...]
    lcnt = jnp.sum((hn[0:1, :] == 0.0).astype(jnp.float32))
    rcnt = jnp.sum((hn[1:2, :] == 0.0).astype(jnp.float32))
    lhanded = lcnt > rcnt

    x = x_ref[...]
    isn = jnp.isnan(x)
    nanany = (isn[0] | isn[1] | isn[2]).astype(jnp.float32)
    x0 = jnp.where(isn, 0.0, x)

    va = jnp.dot(s0_ref[...], x0[0], preferred_element_type=jnp.float32)
    vb = jnp.dot(s1_ref[...], x0[1], preferred_element_type=jnp.float32)
    mk = jnp.dot(sm_ref[...], nanany, preferred_element_type=jnp.float32)

    e = e_ref[...]
    va = jnp.dot(va, e, preferred_element_type=jnp.float32)
    vb = jnp.dot(vb, e, preferred_element_type=jnp.float32)
    mk = jnp.dot(mk, e, preferred_element_type=jnp.float32)

    val0 = jnp.where(lhanded, va[64:128, :], va[0:64, :])
    val1 = jnp.where(lhanded, vb[64:128, :], vb[0:64, :])
    bad = jnp.where(lhanded, mk[64:128, :], mk[0:64, :]) > 0.5
    sgn0 = jnp.where(lhanded, -1.0, 1.0)

    r0 = (val0 - sgn0 * m0) * inv_s
    r1 = (val1 - m1) * inv_s
    r0 = jnp.where(bad, 0.0, r0)
    r1 = jnp.where(bad, 0.0, r1)
    out_ref[0, :, :] = r0
    out_ref[1, :, :] = r1


@jax.jit
def kernel(frames):
    S0, S1, Sm = _build_sel()
    masks = _build_stat_masks()
    xT = jnp.transpose(frames, (2, 1, 0))

    FA = 1024
    partials, handcnt = pl.pallas_call(
        _stats_body,
        grid=(N_FRAMES // FA, _NRB),
        in_specs=[
            pl.BlockSpec((3, _RB, FA), lambda fb, rb: (0, _rowblk(rb), fb)),
            pl.BlockSpec((1, 3, _RB), lambda fb, rb: (rb, 0, 0)),
        ],
        out_specs=[
            pl.BlockSpec((1, 128), lambda fb, rb: (0, 0)),
            pl.BlockSpec((2, FA), lambda fb, rb: (0, fb)),
        ],
        out_shape=[
            jax.ShapeDtypeStruct((1, 128), jnp.float32),
            jax.ShapeDtypeStruct((2, N_FRAMES), jnp.float32),
        ],
    )(xT, masks)

    FB = 512
    E = np.zeros((FB, FB // 2), np.float32)
    E[np.arange(0, FB, 2), np.arange(FB // 2)] = 1.0
    E = jnp.asarray(E)

    full = pl.pallas_call(
        _main_body,
        grid=(N_FRAMES // FB,),
        in_specs=[
            pl.BlockSpec((1, 128), lambda j: (0, 0)),
            pl.BlockSpec((2, N_FRAMES), lambda j: (0, 0)),
            pl.BlockSpec((3, ROWS_PER_FRAME, FB), lambda j: (0, 0, j)),
            pl.BlockSpec((128, ROWS_PER_FRAME), lambda j: (0, 0)),
            pl.BlockSpec((128, ROWS_PER_FRAME), lambda j: (0, 0)),
            pl.BlockSpec((128, ROWS_PER_FRAME), lambda j: (0, 0)),
            pl.BlockSpec((FB, FB // 2), lambda j: (0, 0)),
        ],
        out_specs=pl.BlockSpec((2, 64, FB // 2), lambda j: (0, 0, j)),
        out_shape=jax.ShapeDtypeStruct((2, 64, MAX_LEN), jnp.float32),
    )(partials, handcnt, xT, S0, S1, Sm, E)

    res = full[:, :61, :]
    return jnp.transpose(res, (2, 1, 0))
